# baseline (speedup 1.0000x reference)
import functools

import jax
import jax.numpy as jnp
from jax import lax
from jax.experimental import pallas as pl
from jax.experimental.pallas import tpu as pltpu
from jax.experimental.pallas import tpu_sc as plsc

_NC = 2
_NS = 16
_NW = _NC * _NS
_CH = 32
_NB = 3


@jax.jit
def _sc_gather(idx, table):
    bsz, seq = idx.shape
    _, d = table.shape
    n_rows = bsz * seq
    b_per_w = n_rows // _NW
    w_per_b = seq // b_per_w
    n_chunks = b_per_w // _CH
    mesh = plsc.VectorSubcoreMesh(core_axis_name="c", subcore_axis_name="s")

    @functools.partial(
        pl.kernel,
        out_type=jax.ShapeDtypeStruct((bsz, seq, d), jnp.float32),
        mesh=mesh,
        scratch_types=[
            pltpu.VMEM((b_per_w,), jnp.int32),
            pltpu.VMEM((_NB, _CH, d), jnp.float32),
            pltpu.SemaphoreType.DMA((_NB,)),
            pltpu.SemaphoreType.DMA((_NB,)),
        ],
    )
    def k(idx_hbm, table_hbm, out_hbm, idx_v, bufs, gsem, osem):
        wid = lax.axis_index("s") * _NC + lax.axis_index("c")
        row = wid // w_per_b
        col = (wid % w_per_b) * b_per_w
        pltpu.sync_copy(idx_hbm.at[row, pl.ds(col, b_per_w)], idx_v)

        def gather(c, b):
            return pltpu.async_copy(
                table_hbm.at[idx_v.at[pl.ds(c * _CH, _CH)]], bufs.at[b], gsem.at[b]
            )

        def put(c, b):
            return pltpu.async_copy(
                bufs.at[b], out_hbm.at[row, pl.ds(col + c * _CH, _CH)], osem.at[b]
            )

        gdesc = [None] * _NB
        odesc = [None] * _NB
        gdesc[0] = gather(0, 0)
        for c in range(n_chunks):
            b = c % _NB
            nc = c + 1
            if nc < n_chunks:
                fb = nc % _NB
                if odesc[fb] is not None:
                    odesc[fb].wait()
                gdesc[fb] = gather(nc, fb)
            gdesc[b].wait()
            odesc[b] = put(c, b)
        for c in range(max(0, n_chunks - _NB), n_chunks):
            odesc[c % _NB].wait()

    return k(idx, table)


def kernel(input_ids, shared_weight):
    return _sc_gather(input_ids, shared_weight)


# speedup vs baseline: 1.0140x; 1.0140x over previous
import functools

import jax
import jax.numpy as jnp
from jax import lax
from jax.experimental import pallas as pl
from jax.experimental.pallas import tpu as pltpu
from jax.experimental.pallas import tpu_sc as plsc

_NC = 2
_NS = 16
_NW = _NC * _NS
_CH = 32
_NB = 3


@jax.jit
def _sc_gather(idx, table):
    bsz, seq = idx.shape
    _, d = table.shape
    n_rows = bsz * seq
    b_per_w = n_rows // _NW
    w_per_b = seq // b_per_w
    n_chunks = b_per_w // _CH
    mesh = plsc.VectorSubcoreMesh(core_axis_name="c", subcore_axis_name="s")

    @functools.partial(
        pl.kernel,
        out_type=jax.ShapeDtypeStruct((bsz, seq, d), jnp.float32),
        mesh=mesh,
        scratch_types=[
            pltpu.VMEM((b_per_w,), jnp.int32),
            pltpu.VMEM((_NB, _CH, d), jnp.float32),
            pltpu.SemaphoreType.DMA((_NB,)),
            pltpu.SemaphoreType.DMA((_NB,)),
        ],
    )
    def k(idx_hbm, table_hbm, out_hbm, idx_v, bufs, gsem, osem):
        wid = lax.axis_index("s") * _NC + lax.axis_index("c")
        row = wid // w_per_b
        col = (wid % w_per_b) * b_per_w
        pltpu.sync_copy(idx_hbm.at[row, pl.ds(col, b_per_w)], idx_v)

        def gather(c, b):
            return pltpu.async_copy(
                table_hbm.at[idx_v.at[pl.ds(c * _CH, _CH)]], bufs.at[b], gsem.at[b]
            )

        def put(c, b):
            return pltpu.async_copy(
                bufs.at[b], out_hbm.at[row, pl.ds(col + c * _CH, _CH)], osem.at[b]
            )

        gdesc = [None] * _NB
        odesc = [None] * _NB
        for c in range(min(_NB - 1, n_chunks)):
            gdesc[c % _NB] = gather(c, c % _NB)
        for c in range(n_chunks):
            b = c % _NB
            nc = c + _NB - 1
            if nc < n_chunks:
                fb = nc % _NB
                if odesc[fb] is not None:
                    odesc[fb].wait()
                gdesc[fb] = gather(nc, fb)
            gdesc[b].wait()
            odesc[b] = put(c, b)
        for c in range(max(0, n_chunks - _NB), n_chunks):
            odesc[c % _NB].wait()

    return k(idx, table)


def kernel(input_ids, shared_weight):
    return _sc_gather(input_ids, shared_weight)
